# baseline (speedup 1.0000x reference)
import functools

import jax
import jax.numpy as jnp
from jax import lax
from jax.experimental import pallas as pl
from jax.experimental.pallas import tpu as pltpu
from jax.experimental.pallas import tpu_sc as plsc

DIM = 50
DPAD = 64
PACK = 4
QW = DPAD // 2
BATCH = 16384
GAMMA = 1.0

NC = 2
NS = 16
L = 16
NW = NC * NS
BPW = BATCH // NW
CROWS = 256
NSTEP = 2 * (BPW // CROWS)
NGRP = CROWS // L
IDX_PACK = 6 * BPW


def _sc_kernel(ent_hbm, rel_hbm, idx_hbm, out_hbm,
               idx_v, h_buf, r_buf, t_buf, acc_all, sem):
  wid = lax.axis_index("s") * NC + lax.axis_index("c")
  pltpu.sync_copy(idx_hbm.at[wid], idx_v)

  def issue_chunk(phase, half):
    def issue_grp(g, carry):
      base = phase * (3 * BPW) + half * CROWS + g * L
      ev_h = idx_v[pl.ds(base, L)]
      ev_r = idx_v[pl.ds(base + BPW, L)]
      ev_t = idx_v[pl.ds(base + 2 * BPW, L)]
      for j in range(L):
        i = g * L + j
        pltpu.async_copy(ent_hbm.at[pl.ds(ev_h[j] >> 2, 1)],
                         h_buf.at[pl.ds(i, 1)], sem)
        pltpu.async_copy(rel_hbm.at[pl.ds(ev_r[j] >> 2, 1)],
                         r_buf.at[pl.ds(i, 1)], sem)
        pltpu.async_copy(ent_hbm.at[pl.ds(ev_t[j] >> 2, 1)],
                         t_buf.at[pl.ds(i, 1)], sem)
      return carry
    lax.fori_loop(0, NGRP, issue_grp, jnp.int32(0))

  def drain_chunk():
    pltpu.make_async_copy(ent_hbm.at[pl.ds(0, CROWS)], h_buf, sem).wait()
    pltpu.make_async_copy(ent_hbm.at[pl.ds(0, CROWS)], r_buf, sem).wait()
    pltpu.make_async_copy(ent_hbm.at[pl.ds(0, CROWS)], t_buf, sem).wait()

  def compute_chunk(phase, half):
    def comp_grp(g, carry):
      base = phase * (3 * BPW) + half * CROWS + g * L
      ev_h = idx_v[pl.ds(base, L)]
      ev_r = idx_v[pl.ds(base + BPW, L)]
      ev_t = idx_v[pl.ds(base + 2 * BPW, L)]
      for j in range(L):
        row = g * L + j
        qh = (ev_h[j] & 3) * QW
        qr = (ev_r[j] & 3) * QW
        qt = (ev_t[j] & 3) * QW
        acc = jnp.zeros((L,), jnp.float32)
        for k in range(QW // L):
          hv = plsc.bitcast(h_buf[row, pl.ds(qh + k * L, L)], jnp.bfloat16)
          rv = plsc.bitcast(r_buf[row, pl.ds(qr + k * L, L)], jnp.bfloat16)
          tv = plsc.bitcast(t_buf[row, pl.ds(qt + k * L, L)], jnp.bfloat16)
          e = jnp.abs(hv + rv - tv)
          lo, hi = plsc.unpack(e, format=plsc.PackFormat.INTERLEAVED)
          acc = acc + lo + hi
        off = (phase * BPW + half * CROWS + row) * L
        acc_all[pl.ds(off, L)] = acc
      return carry
    lax.fori_loop(0, NGRP, comp_grp, jnp.int32(0))

  for step in range(NSTEP):
    phase, half = step // 2, step % 2
    issue_chunk(phase, half)
    drain_chunk()
    compute_chunk(phase, half)

  def diff_q(q, carry):
    d = acc_all[pl.ds(q * L, L)] - acc_all[pl.ds(L * BPW + q * L, L)]
    acc_all[pl.ds(q * L, L)] = d
    return carry
  lax.fori_loop(0, BPW, diff_q, jnp.int32(0))
  pltpu.sync_copy(acc_all.at[pl.ds(0, L * BPW)], out_hbm.at[wid])


def _tc_finish_kernel(p_ref, o_ref):
  x = p_ref[...].reshape(NW * BPW * L // 128, 128)
  r0 = lax.broadcasted_iota(jnp.int32, (128, 128 // L), 0) // L
  r1 = lax.broadcasted_iota(jnp.int32, (128, 128 // L), 1)
  m = (r0 == r1).astype(jnp.float32)
  y = lax.dot_general(x, m, (((1,), (0,)), ((), ())),
                      preferred_element_type=jnp.float32)
  o_ref[...] = jnp.sum(
      jnp.maximum(y + jnp.float32(GAMMA), jnp.float32(0.0))
  ).reshape(1, 1)


def _pack_table(tab):
  n = tab.shape[0]
  x = jnp.pad(tab.astype(jnp.bfloat16), ((0, 0), (0, DPAD - DIM)))
  x = x.reshape(n // PACK, PACK * QW, 2)
  return lax.bitcast_convert_type(x, jnp.float32)


@jax.jit
def kernel(pos_head, pos_relation, pos_tail, neg_head, neg_relation, neg_tail,
           entity_emb, relation_emb):
  ent_pk = _pack_table(entity_emb)
  rel_pk = _pack_table(relation_emb)

  packed = jnp.stack([pos_head, pos_relation, pos_tail,
                      neg_head, neg_relation, neg_tail]).astype(jnp.int32)
  packed = packed.reshape(6, NW, BPW).transpose(1, 0, 2).reshape(NW, IDX_PACK)

  mesh = plsc.VectorSubcoreMesh(core_axis_name="c", subcore_axis_name="s")
  sc = pl.kernel(
      _sc_kernel,
      out_type=jax.ShapeDtypeStruct((NW, L * BPW), jnp.float32),
      mesh=mesh,
      compiler_params=pltpu.CompilerParams(needs_layout_passes=False),
      scratch_types=[
          pltpu.VMEM((IDX_PACK,), jnp.int32),
          pltpu.VMEM((CROWS, PACK * QW), jnp.float32),
          pltpu.VMEM((CROWS, PACK * QW), jnp.float32),
          pltpu.VMEM((CROWS, PACK * QW), jnp.float32),
          pltpu.VMEM((2 * L * BPW,), jnp.float32),
          pltpu.SemaphoreType.DMA,
      ],
  )
  partials = sc(ent_pk, rel_pk, packed)

  total = pl.pallas_call(
      _tc_finish_kernel,
      out_shape=jax.ShapeDtypeStruct((1, 1), jnp.float32),
  )(partials)
  return total[0, 0]


# speedup vs baseline: 14.0710x; 14.0710x over previous
import functools

import jax
import jax.numpy as jnp
from jax import lax
from jax.experimental import pallas as pl
from jax.experimental.pallas import tpu as pltpu
from jax.experimental.pallas import tpu_sc as plsc

DIM = 50
QW = 64
PACK = 2
BATCH = 16384
GAMMA = 1.0

NC = 2
NS = 16
L = 16
NW = NC * NS
BPW = BATCH // NW
CROWS = 256
NSTEP = 2 * (BPW // CROWS)
NGRP = CROWS // L
IDX_PACK = 6 * BPW


def _sc_kernel(ent_hbm, rel_hbm, idx_hbm, out_hbm,
               idx_v, h_buf, r_buf, t_buf, acc_all, sem):
  wid = lax.axis_index("s") * NC + lax.axis_index("c")
  pltpu.sync_copy(idx_hbm.at[wid], idx_v)
  iota = lax.iota(jnp.int32, L)

  def issue_chunk(phase, half):
    def issue_grp(g, carry):
      base = phase * (3 * BPW) + half * CROWS + g * L
      ev_h = idx_v[pl.ds(base, L)]
      ev_r = idx_v[pl.ds(base + BPW, L)]
      ev_t = idx_v[pl.ds(base + 2 * BPW, L)]
      for j in range(L):
        i = g * L + j
        pltpu.async_copy(ent_hbm.at[pl.ds(ev_h[j] >> 1, 1)],
                         h_buf.at[pl.ds(i, 1)], sem)
        pltpu.async_copy(rel_hbm.at[pl.ds(ev_r[j] >> 1, 1)],
                         r_buf.at[pl.ds(i, 1)], sem)
        pltpu.async_copy(ent_hbm.at[pl.ds(ev_t[j] >> 1, 1)],
                         t_buf.at[pl.ds(i, 1)], sem)
      return carry
    lax.fori_loop(0, NGRP, issue_grp, jnp.int32(0))

  def drain_chunk():
    pltpu.make_async_copy(ent_hbm.at[pl.ds(0, CROWS)], h_buf, sem).wait()
    pltpu.make_async_copy(ent_hbm.at[pl.ds(0, CROWS)], r_buf, sem).wait()
    pltpu.make_async_copy(ent_hbm.at[pl.ds(0, CROWS)], t_buf, sem).wait()

  def compute_chunk(phase, half):
    def comp_grp(g, carry):
      base = phase * (3 * BPW) + half * CROWS + g * L
      ev_h = idx_v[pl.ds(base, L)]
      ev_r = idx_v[pl.ds(base + BPW, L)]
      ev_t = idx_v[pl.ds(base + 2 * BPW, L)]
      for j in range(L):
        row = jnp.full((L,), g * L + j, jnp.int32)
        qh = (ev_h[j] & 1) * QW + iota
        qr = (ev_r[j] & 1) * QW + iota
        qt = (ev_t[j] & 1) * QW + iota
        acc = jnp.zeros((L,), jnp.float32)
        for k in range(QW // L):
          hv = plsc.load_gather(h_buf, [row, qh + (k * L)])
          rv = plsc.load_gather(r_buf, [row, qr + (k * L)])
          tv = plsc.load_gather(t_buf, [row, qt + (k * L)])
          acc = acc + jnp.abs(hv + rv - tv)
        off = (phase * BPW + half * CROWS + g * L + j) * L
        acc_all[pl.ds(off, L)] = acc
      return carry
    lax.fori_loop(0, NGRP, comp_grp, jnp.int32(0))

  for step in range(NSTEP):
    phase, half = step // 2, step % 2
    issue_chunk(phase, half)
    drain_chunk()
    compute_chunk(phase, half)

  def diff_q(q, carry):
    d = acc_all[pl.ds(q * L, L)] - acc_all[pl.ds(L * BPW + q * L, L)]
    acc_all[pl.ds(q * L, L)] = d
    return carry
  lax.fori_loop(0, BPW, diff_q, jnp.int32(0))
  pltpu.sync_copy(acc_all.at[pl.ds(0, L * BPW)], out_hbm.at[wid])


def _tc_finish_kernel(p_ref, o_ref):
  x = p_ref[...].reshape(NW * BPW * L // 128, 128)
  r0 = lax.broadcasted_iota(jnp.int32, (128, 128 // L), 0) // L
  r1 = lax.broadcasted_iota(jnp.int32, (128, 128 // L), 1)
  m = (r0 == r1).astype(jnp.float32)
  y = lax.dot_general(x, m, (((1,), (0,)), ((), ())),
                      preferred_element_type=jnp.float32)
  o_ref[...] = jnp.sum(
      jnp.maximum(y + jnp.float32(GAMMA), jnp.float32(0.0))
  ).reshape(1, 1)


def _pack_table(tab):
  n = tab.shape[0]
  x = jnp.pad(tab, ((0, 0), (0, QW - DIM)))
  return x.reshape(n // PACK, PACK * QW)


@jax.jit
def kernel(pos_head, pos_relation, pos_tail, neg_head, neg_relation, neg_tail,
           entity_emb, relation_emb):
  ent_pk = _pack_table(entity_emb)
  rel_pk = _pack_table(relation_emb)

  packed = jnp.stack([pos_head, pos_relation, pos_tail,
                      neg_head, neg_relation, neg_tail]).astype(jnp.int32)
  packed = packed.reshape(6, NW, BPW).transpose(1, 0, 2).reshape(NW, IDX_PACK)

  mesh = plsc.VectorSubcoreMesh(core_axis_name="c", subcore_axis_name="s")
  sc = pl.kernel(
      _sc_kernel,
      out_type=jax.ShapeDtypeStruct((NW, L * BPW), jnp.float32),
      mesh=mesh,
      compiler_params=pltpu.CompilerParams(needs_layout_passes=False),
      scratch_types=[
          pltpu.VMEM((IDX_PACK,), jnp.int32),
          pltpu.VMEM((CROWS, PACK * QW), jnp.float32),
          pltpu.VMEM((CROWS, PACK * QW), jnp.float32),
          pltpu.VMEM((CROWS, PACK * QW), jnp.float32),
          pltpu.VMEM((2 * L * BPW,), jnp.float32),
          pltpu.SemaphoreType.DMA,
      ],
  )
  partials = sc(ent_pk, rel_pk, packed)

  total = pl.pallas_call(
      _tc_finish_kernel,
      out_shape=jax.ShapeDtypeStruct((1, 1), jnp.float32),
  )(partials)
  return total[0, 0]


# speedup vs baseline: 17.5546x; 1.2476x over previous
import functools

import jax
import jax.numpy as jnp
from jax import lax
from jax.experimental import pallas as pl
from jax.experimental.pallas import tpu as pltpu
from jax.experimental.pallas import tpu_sc as plsc

DIM = 50
PACK = 2
PW = PACK * DIM
BATCH = 16384
GAMMA = 1.0

NC = 2
NS = 16
L = 16
NW = NC * NS
BPW = BATCH // NW
CROWS = 256
NSTEP = 2 * (BPW // CROWS)
NGRP = CROWS // L
IDX_PACK = 6 * BPW


def _sc_kernel(ent_hbm, rel_hbm, idx_hbm, out_hbm,
               idx_v, h_buf, r_buf, t_buf, acc_all, sem):
  wid = lax.axis_index("s") * NC + lax.axis_index("c")
  pltpu.sync_copy(idx_hbm.at[wid], idx_v)
  iota = lax.iota(jnp.int32, L)
  tail_mask = iota >= (4 * L - DIM)

  def issue_chunk(phase, half):
    def issue_grp(g, carry):
      base = phase * (3 * BPW) + half * CROWS + g * L
      ev_h = idx_v[pl.ds(base, L)]
      ev_r = idx_v[pl.ds(base + BPW, L)]
      ev_t = idx_v[pl.ds(base + 2 * BPW, L)]
      for j in range(L):
        i = g * L + j
        pltpu.async_copy(ent_hbm.at[pl.ds(ev_h[j] >> 1, 1)],
                         h_buf.at[pl.ds(i, 1)], sem)
        pltpu.async_copy(rel_hbm.at[pl.ds(ev_r[j] >> 1, 1)],
                         r_buf.at[pl.ds(i, 1)], sem)
        pltpu.async_copy(ent_hbm.at[pl.ds(ev_t[j] >> 1, 1)],
                         t_buf.at[pl.ds(i, 1)], sem)
      return carry
    lax.fori_loop(0, NGRP, issue_grp, jnp.int32(0))

  def drain_chunk():
    pltpu.make_async_copy(ent_hbm.at[pl.ds(0, CROWS)], h_buf, sem).wait()
    pltpu.make_async_copy(ent_hbm.at[pl.ds(0, CROWS)], r_buf, sem).wait()
    pltpu.make_async_copy(ent_hbm.at[pl.ds(0, CROWS)], t_buf, sem).wait()

  def compute_chunk(phase, half):
    def comp_grp(g, carry):
      base = phase * (3 * BPW) + half * CROWS + g * L
      ev_h = idx_v[pl.ds(base, L)]
      ev_r = idx_v[pl.ds(base + BPW, L)]
      ev_t = idx_v[pl.ds(base + 2 * BPW, L)]
      for j in range(L):
        row = jnp.full((L,), g * L + j, jnp.int32)
        qh = (ev_h[j] & 1) * DIM + iota
        qr = (ev_r[j] & 1) * DIM + iota
        qt = (ev_t[j] & 1) * DIM + iota
        acc = jnp.zeros((L,), jnp.float32)
        for k in range(3):
          hv = plsc.load_gather(h_buf, [row, qh + (k * L)])
          rv = plsc.load_gather(r_buf, [row, qr + (k * L)])
          tv = plsc.load_gather(t_buf, [row, qt + (k * L)])
          acc = acc + jnp.abs(hv + rv - tv)
        hv = plsc.load_gather(h_buf, [row, qh + (DIM - L)])
        rv = plsc.load_gather(r_buf, [row, qr + (DIM - L)])
        tv = plsc.load_gather(t_buf, [row, qt + (DIM - L)])
        e = jnp.abs(hv + rv - tv)
        acc = acc + jnp.where(tail_mask, e, jnp.float32(0.0))
        off = (phase * BPW + half * CROWS + g * L + j) * L
        acc_all[pl.ds(off, L)] = acc
      return carry
    lax.fori_loop(0, NGRP, comp_grp, jnp.int32(0))

  for step in range(NSTEP):
    phase, half = step // 2, step % 2
    issue_chunk(phase, half)
    drain_chunk()
    compute_chunk(phase, half)

  def diff_q(q, carry):
    d = acc_all[pl.ds(q * L, L)] - acc_all[pl.ds(L * BPW + q * L, L)]
    acc_all[pl.ds(q * L, L)] = d
    return carry
  lax.fori_loop(0, BPW, diff_q, jnp.int32(0))
  pltpu.sync_copy(acc_all.at[pl.ds(0, L * BPW)], out_hbm.at[wid])


def _tc_finish_kernel(p_ref, o_ref):
  x = p_ref[...].reshape(NW * BPW * L // 128, 128)
  r0 = lax.broadcasted_iota(jnp.int32, (128, 128 // L), 0) // L
  r1 = lax.broadcasted_iota(jnp.int32, (128, 128 // L), 1)
  m = (r0 == r1).astype(jnp.float32)
  y = lax.dot_general(x, m, (((1,), (0,)), ((), ())),
                      preferred_element_type=jnp.float32)
  o_ref[...] = jnp.sum(
      jnp.maximum(y + jnp.float32(GAMMA), jnp.float32(0.0))
  ).reshape(1, 1)


@jax.jit
def kernel(pos_head, pos_relation, pos_tail, neg_head, neg_relation, neg_tail,
           entity_emb, relation_emb):
  ent_pk = entity_emb.reshape(entity_emb.shape[0] // PACK, PW)
  rel_pk = relation_emb.reshape(relation_emb.shape[0] // PACK, PW)

  packed = jnp.stack([pos_head, pos_relation, pos_tail,
                      neg_head, neg_relation, neg_tail]).astype(jnp.int32)
  packed = packed.reshape(6, NW, BPW).transpose(1, 0, 2).reshape(NW, IDX_PACK)

  mesh = plsc.VectorSubcoreMesh(core_axis_name="c", subcore_axis_name="s")
  sc = pl.kernel(
      _sc_kernel,
      out_type=jax.ShapeDtypeStruct((NW, L * BPW), jnp.float32),
      mesh=mesh,
      compiler_params=pltpu.CompilerParams(needs_layout_passes=False),
      scratch_types=[
          pltpu.VMEM((IDX_PACK,), jnp.int32),
          pltpu.VMEM((CROWS, PW), jnp.float32),
          pltpu.VMEM((CROWS, PW), jnp.float32),
          pltpu.VMEM((CROWS, PW), jnp.float32),
          pltpu.VMEM((2 * L * BPW,), jnp.float32),
          pltpu.SemaphoreType.DMA,
      ],
  )
  partials = sc(ent_pk, rel_pk, packed)

  total = pl.pallas_call(
      _tc_finish_kernel,
      out_shape=jax.ShapeDtypeStruct((1, 1), jnp.float32),
  )(partials)
  return total[0, 0]


# speedup vs baseline: 56.7945x; 3.2353x over previous
import functools

import jax
import jax.numpy as jnp
from jax import lax
from jax.experimental import pallas as pl
from jax.experimental.pallas import tpu as pltpu
from jax.experimental.pallas import tpu_sc as plsc

DIM = 50
BATCH = 16384
GAMMA = 1.0

NC = 2
NS = 16
L = 16
NW = NC * NS
BPW = BATCH // NW
CROWS = 256
NSTEP = 2 * (BPW // CROWS)
NGRP = CROWS // L
IDX_PACK = 6 * BPW


def _sc_kernel(ent_hbm, rel_hbm, idx_hbm, out_hbm,
               idx_v, h_buf, r_buf, t_buf, acc_all, sem):
  wid = lax.axis_index("s") * NC + lax.axis_index("c")
  pltpu.sync_copy(idx_hbm.at[wid], idx_v)
  iota = lax.iota(jnp.int32, L)
  tail_mask = iota >= (4 * L - DIM)

  def issue_chunk(phase, half):
    def issue_grp(g, carry):
      base = phase * (3 * BPW) + half * CROWS + g * L
      ev_h = idx_v[pl.ds(base, L)]
      ev_r = idx_v[pl.ds(base + BPW, L)]
      ev_t = idx_v[pl.ds(base + 2 * BPW, L)]
      for j in range(L):
        row = g * L + j
        pltpu.async_copy(ent_hbm.at[pl.ds(ev_h[j], 1)],
                         h_buf.at[pl.ds(row, 1)], sem)
        pltpu.async_copy(rel_hbm.at[pl.ds(ev_r[j], 1)],
                         r_buf.at[pl.ds(row, 1)], sem)
        pltpu.async_copy(ent_hbm.at[pl.ds(ev_t[j], 1)],
                         t_buf.at[pl.ds(row, 1)], sem)
      return carry
    lax.fori_loop(0, NGRP, issue_grp, jnp.int32(0))

  def drain_chunk():
    pltpu.make_async_copy(ent_hbm.at[pl.ds(0, CROWS)], h_buf, sem).wait()
    pltpu.make_async_copy(ent_hbm.at[pl.ds(0, CROWS)], r_buf, sem).wait()
    pltpu.make_async_copy(ent_hbm.at[pl.ds(0, CROWS)], t_buf, sem).wait()

  def compute_chunk(phase, half):
    def comp_grp(g, carry):
      for j in range(L):
        row = g * L + j
        acc = jnp.zeros((L,), jnp.float32)
        for k in range(3):
          sl = pl.ds(k * L, L)
          e = jnp.abs(h_buf[row, sl] + r_buf[row, sl] - t_buf[row, sl])
          acc = acc + e
        sl = pl.ds(DIM - L, L)
        e = jnp.abs(h_buf[row, sl] + r_buf[row, sl] - t_buf[row, sl])
        acc = acc + jnp.where(tail_mask, e, jnp.float32(0.0))
        off = phase * (L * BPW) + (half * CROWS + row) * L
        acc_all[pl.ds(off, L)] = acc
      return carry
    lax.fori_loop(0, NGRP, comp_grp, jnp.int32(0))

  for step in range(NSTEP):
    phase, half = step // 2, step % 2
    issue_chunk(phase, half)
    drain_chunk()
    compute_chunk(phase, half)

  def diff_q(q, carry):
    d = acc_all[pl.ds(q * L, L)] - acc_all[pl.ds(L * BPW + q * L, L)]
    acc_all[pl.ds(q * L, L)] = d
    return carry
  lax.fori_loop(0, BPW, diff_q, jnp.int32(0))
  pltpu.sync_copy(acc_all.at[pl.ds(0, L * BPW)], out_hbm.at[wid])


def _tc_finish_kernel(p_ref, o_ref):
  x = p_ref[...].reshape(NW * BPW * L // 128, 128)
  r0 = lax.broadcasted_iota(jnp.int32, (128, 128 // L), 0) // L
  r1 = lax.broadcasted_iota(jnp.int32, (128, 128 // L), 1)
  m = (r0 == r1).astype(jnp.float32)
  y = lax.dot_general(x, m, (((1,), (0,)), ((), ())),
                      preferred_element_type=jnp.float32)
  o_ref[...] = jnp.sum(
      jnp.maximum(y + jnp.float32(GAMMA), jnp.float32(0.0))
  ).reshape(1, 1)


@jax.jit
def kernel(pos_head, pos_relation, pos_tail, neg_head, neg_relation, neg_tail,
           entity_emb, relation_emb):
  packed = jnp.stack([pos_head, pos_relation, pos_tail,
                      neg_head, neg_relation, neg_tail]).astype(jnp.int32)
  packed = packed.reshape(6, NW, BPW).transpose(1, 0, 2).reshape(NW, IDX_PACK)

  mesh = plsc.VectorSubcoreMesh(core_axis_name="c", subcore_axis_name="s")
  sc = pl.kernel(
      _sc_kernel,
      out_type=jax.ShapeDtypeStruct((NW, L * BPW), jnp.float32),
      mesh=mesh,
      compiler_params=pltpu.CompilerParams(needs_layout_passes=False),
      scratch_types=[
          pltpu.VMEM((IDX_PACK,), jnp.int32),
          pltpu.VMEM((CROWS, DIM), jnp.float32),
          pltpu.VMEM((CROWS, DIM), jnp.float32),
          pltpu.VMEM((CROWS, DIM), jnp.float32),
          pltpu.VMEM((2 * L * BPW,), jnp.float32),
          pltpu.SemaphoreType.DMA,
      ],
  )
  partials = sc(entity_emb, relation_emb, packed)

  total = pl.pallas_call(
      _tc_finish_kernel,
      out_shape=jax.ShapeDtypeStruct((1, 1), jnp.float32),
  )(partials)
  return total[0, 0]
